# baseline (speedup 1.0000x reference)
import functools

import jax
import jax.numpy as jnp
from jax import lax
from jax.experimental import pallas as pl
from jax.experimental.pallas import tpu as pltpu
from jax.experimental.pallas import tpu_sc as plsc

N_NODES = 10000
N_EDGES = 320000
HIDDEN = 128
EDGE_DIM = 16
NUM_LAYERS = 3

NC = 2
NS = 16
LANES = 16

HALF = HIDDEN // NC
EDGES_PER_TILE = N_EDGES // NS
CHUNK = 80
CHUNKS_PER_TILE = EDGES_PER_TILE // CHUNK
NPAD = 10240
ROWS_PER_TILE = NPAD // NS
GROUPS = HALF // LANES


def _ea_proj_body(attr_ref, we_ref, be_ref, out_ref):
    out_ref[...] = (
        jnp.dot(attr_ref[...], we_ref[...], preferred_element_type=jnp.float32)
        + be_ref[0]
    )


def _project_edge_attr(edge_attr, We_l, be_l):
    eb = 16000
    return pl.pallas_call(
        _ea_proj_body,
        grid=(N_EDGES // eb,),
        in_specs=[
            pl.BlockSpec((eb, EDGE_DIM), lambda i: (i, 0)),
            pl.BlockSpec((EDGE_DIM, HIDDEN), lambda i: (0, 0)),
            pl.BlockSpec((1, HIDDEN), lambda i: (0, 0)),
        ],
        out_specs=pl.BlockSpec((eb, HIDDEN), lambda i: (i, 0)),
        out_shape=jax.ShapeDtypeStruct((N_EDGES, HIDDEN), jnp.float32),
    )(edge_attr, We_l, be_l.reshape(1, HIDDEN))


def _sc_aggregate_body(xh_hbm, ea_hbm, src_hbm, dst_hbm, out_hbm,
                       src0, dst0, sdst0, xrows0, msg0,
                       src1, dst1, sdst1, xrows1, msg1,
                       zero_v, aggr_sh,
                       isem0, isem1, dsem0, dsem1, ssem0, ssem1):
    cid = lax.axis_index("c")
    sid = lax.axis_index("s")
    col = cid * HALF

    def _zero_group(j, c):
        r = j // GROUPS
        g = j % GROUPS
        zero_v[r, pl.ds(g * LANES, LANES)] = jnp.zeros((LANES,), jnp.float32)
        return c
    lax.fori_loop(0, ROWS_PER_TILE * GROUPS, _zero_group, 0)
    pltpu.sync_copy(zero_v, aggr_sh.at[pl.ds(sid * ROWS_PER_TILE, ROWS_PER_TILE)])
    plsc.subcore_barrier()

    base0 = sid * EDGES_PER_TILE
    sets = ((src0, dst0, sdst0, xrows0, msg0, isem0, dsem0, ssem0),
            (src1, dst1, sdst1, xrows1, msg1, isem1, dsem1, ssem1))

    def issue_idx(c, S):
        b = base0 + c * CHUNK
        pltpu.async_copy(src_hbm.at[pl.ds(b, CHUNK)], S[0], S[5])
        pltpu.async_copy(dst_hbm.at[pl.ds(b, CHUNK)], S[1], S[5])

    def wait_idx(S):
        pltpu.make_async_copy(src_hbm.at[pl.ds(0, CHUNK)], S[0], S[5]).wait()
        pltpu.make_async_copy(dst_hbm.at[pl.ds(0, CHUNK)], S[1], S[5]).wait()

    def issue_data(c, S):
        b = base0 + c * CHUNK
        pltpu.async_copy(ea_hbm.at[pl.ds(b, CHUNK), pl.ds(col, HALF)], S[4], S[6])
        pltpu.async_copy(xh_hbm.at[cid, pl.ds(0, CHUNK)], S[3], S[6])

    def wait_data(S):
        pltpu.make_async_copy(ea_hbm.at[pl.ds(0, CHUNK), pl.ds(col, HALF)],
                              S[4], S[6]).wait()
        pltpu.make_async_copy(ea_hbm.at[pl.ds(0, CHUNK), pl.ds(col, HALF)],
                              S[3], S[6]).wait()

    def issue_scatter(S):
        pltpu.async_copy(S[4], aggr_sh.at[S[2]], S[7], add=True)

    def wait_scatter(S):
        pltpu.make_async_copy(ea_hbm.at[pl.ds(0, CHUNK), pl.ds(col, HALF)],
                              S[4], S[7]).wait()

    def compute(S):
        def _edge(e, c):
            for g in range(GROUPS):
                sl = pl.ds(g * LANES, LANES)
                S[4][e, sl] = jnp.maximum(S[3][e, sl] + S[4][e, sl], 0.0)
            return c
        lax.fori_loop(0, CHUNK, _edge, 0)
        for g in range(CHUNK // LANES):
            sl = pl.ds(g * LANES, LANES)
            S[2][sl] = S[1][sl]

    S0, S1 = sets
    pltpu.sync_copy(src_hbm.at[pl.ds(base0, CHUNK)], S0[0])
    pltpu.sync_copy(dst_hbm.at[pl.ds(base0, CHUNK)], S0[1])
    issue_data(0, S0)
    issue_idx(1, S1)

    NP2 = CHUNKS_PER_TILE // 2

    def _pair(i2, carry):
        c = 2 * i2
        wait_data(S0)
        wait_idx(S1)

        @pl.when(i2 > 0)
        def _():
            wait_scatter(S1)
        issue_data(c + 1, S1)
        compute(S0)
        issue_scatter(S0)

        @pl.when(i2 < NP2 - 1)
        def _():
            issue_idx(c + 2, S0)

        wait_data(S1)
        wait_scatter(S0)

        @pl.when(i2 < NP2 - 1)
        def _():
            wait_idx(S0)
            issue_data(c + 2, S0)
        compute(S1)
        issue_scatter(S1)

        @pl.when(i2 < NP2 - 1)
        def _():
            issue_idx(c + 3, S1)
        return carry

    lax.fori_loop(0, NP2, _pair, 0)
    wait_scatter(S1)
    plsc.subcore_barrier()

    pltpu.sync_copy(aggr_sh.at[pl.ds(sid * ROWS_PER_TILE, ROWS_PER_TILE)], zero_v)
    pltpu.sync_copy(zero_v,
                    out_hbm.at[pl.ds(sid * ROWS_PER_TILE, ROWS_PER_TILE),
                               pl.ds(col, HALF)])


@functools.cache
def _sc_aggregate_kernel():
  idx_t = pltpu.VMEM((CHUNK,), jnp.int32)
  row_t = pltpu.VMEM((CHUNK, HALF), jnp.float32)
  return pl.kernel(
    _sc_aggregate_body,
    out_type=jax.ShapeDtypeStruct((NPAD, HIDDEN), jnp.float32),
    mesh=plsc.VectorSubcoreMesh(core_axis_name="c", subcore_axis_name="s",
                                num_cores=NC, num_subcores=NS),
    scratch_types=[
        idx_t, idx_t, idx_t, row_t, row_t,
        idx_t, idx_t, idx_t, row_t, row_t,
        pltpu.VMEM((ROWS_PER_TILE, HALF), jnp.float32),
        pltpu.VMEM_SHARED((NPAD, HALF), jnp.float32),
        pltpu.SemaphoreType.DMA, pltpu.SemaphoreType.DMA,
        pltpu.SemaphoreType.DMA, pltpu.SemaphoreType.DMA,
        pltpu.SemaphoreType.DMA, pltpu.SemaphoreType.DMA,
    ],
    compiler_params=pltpu.CompilerParams(use_tc_tiling_on_sc=False),
  )


def _split_body(x_ref, out_ref):
    out_ref[0] = x_ref[:, :HALF]
    out_ref[1] = x_ref[:, HALF:]


def _split_halves(x):
    nb = 2000
    return pl.pallas_call(
        _split_body,
        grid=(N_NODES // nb,),
        in_specs=[pl.BlockSpec((nb, HIDDEN), lambda i: (i, 0))],
        out_specs=pl.BlockSpec((NC, nb, HALF), lambda i: (0, i, 0)),
        out_shape=jax.ShapeDtypeStruct((NC, N_NODES, HALF), jnp.float32),
    )(x)


def _extract_body(ei_ref, src_ref, dst_ref):
    src_ref[...] = ei_ref[0]
    dst_ref[...] = ei_ref[1]


def _extract_indices(ei):
    return pl.pallas_call(
        _extract_body,
        out_shape=[jax.ShapeDtypeStruct((N_EDGES,), jnp.int32),
                   jax.ShapeDtypeStruct((N_EDGES,), jnp.int32)],
    )(ei)


def _node_body(x_ref, p_ref, w1_ref, b1_ref, w2_ref, b2_ref, g_ref, bl_ref,
               out_ref, outh_ref):
    x = x_ref[...]
    h = x + p_ref[...]
    t = jnp.dot(h, w1_ref[...], preferred_element_type=jnp.float32) + b1_ref[0]
    t = t * jax.nn.sigmoid(t)
    t = jnp.dot(t, w2_ref[...], preferred_element_type=jnp.float32) + b2_ref[0]
    y = x + t
    mu = jnp.mean(y, axis=1, keepdims=True)
    var = jnp.mean((y - mu) ** 2, axis=1, keepdims=True)
    y = (y - mu) * lax.rsqrt(var + 1e-5) * g_ref[0] + bl_ref[0]
    y = y * jax.nn.sigmoid(y)
    out_ref[...] = y
    outh_ref[0] = y[:, :HALF]
    outh_ref[1] = y[:, HALF:]


def _node_update(x, parts, w1, b1, w2, b2, g, bl):
    nb = 2000
    grid = (N_NODES // nb,)
    vec = lambda a: a.reshape(1, HIDDEN)
    return pl.pallas_call(
        _node_body,
        grid=grid,
        in_specs=[
            pl.BlockSpec((nb, HIDDEN), lambda i: (i, 0)),
            pl.BlockSpec((nb, HIDDEN), lambda i: (i, 0)),
            pl.BlockSpec((HIDDEN, HIDDEN), lambda i: (0, 0)),
            pl.BlockSpec((1, HIDDEN), lambda i: (0, 0)),
            pl.BlockSpec((HIDDEN, HIDDEN), lambda i: (0, 0)),
            pl.BlockSpec((1, HIDDEN), lambda i: (0, 0)),
            pl.BlockSpec((1, HIDDEN), lambda i: (0, 0)),
            pl.BlockSpec((1, HIDDEN), lambda i: (0, 0)),
        ],
        out_specs=[pl.BlockSpec((nb, HIDDEN), lambda i: (i, 0)),
                   pl.BlockSpec((NC, nb, HALF), lambda i: (0, i, 0))],
        out_shape=[jax.ShapeDtypeStruct((N_NODES, HIDDEN), jnp.float32),
                   jax.ShapeDtypeStruct((NC, N_NODES, HALF), jnp.float32)],
    )(x, parts, w1, vec(b1), w2, vec(b2), vec(g), vec(bl))


def kernel(s, edge_index_bond, edge_attr_bond, W1, b1, W2, b2, We, be,
           ln_g, ln_b):
    src, dst = _extract_indices(edge_index_bond.astype(jnp.int32))
    x = s
    xh = _split_halves(s)
    for l in range(NUM_LAYERS):
        ea = _project_edge_attr(edge_attr_bond, We[l], be[l])
        parts = _sc_aggregate_kernel()(xh, ea, src, dst)
        x, xh = _node_update(x, parts, W1[l], b1[l], W2[l], b2[l],
                             ln_g[l], ln_b[l])
    return x


# speedup vs baseline: 1.8869x; 1.8869x over previous
import functools

import jax
import jax.numpy as jnp
from jax import lax
from jax.experimental import pallas as pl
from jax.experimental.pallas import tpu as pltpu
from jax.experimental.pallas import tpu_sc as plsc

N_NODES = 10000
N_EDGES = 320000
HIDDEN = 128
EDGE_DIM = 16
NUM_LAYERS = 3

NC = 2
NS = 16
LANES = 16

HALF = HIDDEN // NC
CHUNK = 128
N_CHUNKS = N_EDGES // CHUNK
CHUNKS_PER_TILE = N_CHUNKS // NS
N_EXTRA = N_CHUNKS - CHUNKS_PER_TILE * NS
NPAD = 10240
ROWS_PER_TILE = NPAD // NS
GROUPS = HALF // LANES


def _ea_proj_body(attr_ref, we_ref, be_ref, out_ref):
    out_ref[...] = lax.dot_general(
        attr_ref[...], we_ref[...], (((0,), (0,)), ((), ())),
        preferred_element_type=jnp.float32) + be_ref[0]


def _project_edge_attr(edge_attr_t, We_l, be_l):
    eb = 16000
    return pl.pallas_call(
        _ea_proj_body,
        grid=(N_EDGES // eb,),
        in_specs=[
            pl.BlockSpec((EDGE_DIM, eb), lambda i: (0, i)),
            pl.BlockSpec((EDGE_DIM, HIDDEN), lambda i: (0, 0)),
            pl.BlockSpec((1, HIDDEN), lambda i: (0, 0)),
        ],
        out_specs=pl.BlockSpec((eb, HIDDEN), lambda i: (i, 0)),
        out_shape=jax.ShapeDtypeStruct((N_EDGES, HIDDEN), jnp.float32),
    )(edge_attr_t, We_l, be_l.reshape(1, HIDDEN))


def _sc_aggregate_body(xh_hbm, ea_hbm, src_hbm, dst_hbm, out_hbm,
                       src0, dst0, sdst0, xrows0, msg0,
                       src1, dst1, sdst1, xrows1, msg1,
                       zero_v, aggr_sh,
                       isem0, isem1, dsem0, dsem1, ssem0, ssem1):
    cid = lax.axis_index("c")
    sid = lax.axis_index("s")
    col = cid * HALF

    def _zero_group(j, c):
        r = j // GROUPS
        g = j % GROUPS
        zero_v[r, pl.ds(g * LANES, LANES)] = jnp.zeros((LANES,), jnp.float32)
        return c
    lax.fori_loop(0, ROWS_PER_TILE * GROUPS, _zero_group, 0)
    pltpu.sync_copy(zero_v, aggr_sh.at[pl.ds(sid * ROWS_PER_TILE, ROWS_PER_TILE)])
    plsc.subcore_barrier()

    base0 = sid * CHUNKS_PER_TILE * CHUNK
    sets = ((src0, dst0, sdst0, xrows0, msg0, isem0, dsem0, ssem0),
            (src1, dst1, sdst1, xrows1, msg1, isem1, dsem1, ssem1))

    def issue_idx(c, S):
        b = base0 + c * CHUNK
        pltpu.async_copy(src_hbm.at[pl.ds(b, CHUNK)], S[0], S[5])
        pltpu.async_copy(dst_hbm.at[pl.ds(b, CHUNK)], S[1], S[5])

    def wait_idx(S):
        pltpu.make_async_copy(src_hbm.at[pl.ds(0, CHUNK)], S[0], S[5]).wait()
        pltpu.make_async_copy(dst_hbm.at[pl.ds(0, CHUNK)], S[1], S[5]).wait()

    def issue_data(c, S):
        b = base0 + c * CHUNK
        pltpu.async_copy(ea_hbm.at[pl.ds(b, CHUNK), pl.ds(col, HALF)], S[4], S[6])
        pltpu.async_copy(xh_hbm.at[cid].at[S[0]], S[3], S[6])

    def wait_data(S):
        pltpu.make_async_copy(ea_hbm.at[pl.ds(0, CHUNK), pl.ds(col, HALF)],
                              S[4], S[6]).wait()
        pltpu.make_async_copy(ea_hbm.at[pl.ds(0, CHUNK), pl.ds(col, HALF)],
                              S[3], S[6]).wait()

    def issue_scatter(S):
        pltpu.async_copy(S[4], aggr_sh.at[S[2]], S[7], add=True)

    def wait_scatter(S):
        pltpu.make_async_copy(ea_hbm.at[pl.ds(0, CHUNK), pl.ds(col, HALF)],
                              S[4], S[7]).wait()

    def compute(S):
        def _edge(e, c):
            for g in range(GROUPS):
                sl = pl.ds(g * LANES, LANES)
                S[4][e, sl] = jnp.maximum(S[3][e, sl] + S[4][e, sl], 0.0)
            return c
        lax.fori_loop(0, CHUNK, _edge, 0)
        for g in range(CHUNK // LANES):
            sl = pl.ds(g * LANES, LANES)
            S[2][sl] = S[1][sl]

    S0, S1 = sets
    pltpu.sync_copy(src_hbm.at[pl.ds(base0, CHUNK)], S0[0])
    pltpu.sync_copy(dst_hbm.at[pl.ds(base0, CHUNK)], S0[1])
    issue_data(0, S0)
    issue_idx(1, S1)

    NP2 = CHUNKS_PER_TILE // 2

    def _pair(i2, carry):
        c = 2 * i2
        wait_data(S0)
        wait_idx(S1)

        @pl.when(i2 > 0)
        def _():
            wait_scatter(S1)
        issue_data(c + 1, S1)
        compute(S0)
        issue_scatter(S0)

        @pl.when(i2 < NP2 - 1)
        def _():
            issue_idx(c + 2, S0)

        wait_data(S1)
        wait_scatter(S0)

        @pl.when(i2 < NP2 - 1)
        def _():
            wait_idx(S0)
            issue_data(c + 2, S0)
        compute(S1)
        issue_scatter(S1)

        @pl.when(i2 < NP2 - 1)
        def _():
            issue_idx(c + 3, S1)
        return carry

    lax.fori_loop(0, NP2, _pair, 0)

    @pl.when(sid < N_EXTRA)
    def _():
        eb0 = (NS * CHUNKS_PER_TILE + sid) * CHUNK
        pltpu.sync_copy(src_hbm.at[pl.ds(eb0, CHUNK)], S0[0])
        pltpu.sync_copy(dst_hbm.at[pl.ds(eb0, CHUNK)], S0[1])
        pltpu.sync_copy(ea_hbm.at[pl.ds(eb0, CHUNK), pl.ds(col, HALF)], S0[4])
        pltpu.async_copy(xh_hbm.at[cid].at[S0[0]], S0[3], S0[6]).wait()
        compute(S0)
        issue_scatter(S0)
        wait_scatter(S0)

    wait_scatter(S1)
    plsc.subcore_barrier()

    pltpu.sync_copy(aggr_sh.at[pl.ds(sid * ROWS_PER_TILE, ROWS_PER_TILE)], zero_v)
    pltpu.sync_copy(zero_v,
                    out_hbm.at[pl.ds(sid * ROWS_PER_TILE, ROWS_PER_TILE),
                               pl.ds(col, HALF)])


@functools.cache
def _sc_aggregate_kernel():
  idx_t = pltpu.VMEM((CHUNK,), jnp.int32)
  row_t = pltpu.VMEM((CHUNK, HALF), jnp.float32)
  return pl.kernel(
    _sc_aggregate_body,
    out_type=jax.ShapeDtypeStruct((NPAD, HIDDEN), jnp.float32),
    mesh=plsc.VectorSubcoreMesh(core_axis_name="c", subcore_axis_name="s",
                                num_cores=NC, num_subcores=NS),
    scratch_types=[
        idx_t, idx_t, idx_t, row_t, row_t,
        idx_t, idx_t, idx_t, row_t, row_t,
        pltpu.VMEM((ROWS_PER_TILE, HALF), jnp.float32),
        pltpu.VMEM_SHARED((NPAD, HALF), jnp.float32),
        pltpu.SemaphoreType.DMA, pltpu.SemaphoreType.DMA,
        pltpu.SemaphoreType.DMA, pltpu.SemaphoreType.DMA,
        pltpu.SemaphoreType.DMA, pltpu.SemaphoreType.DMA,
    ],
    compiler_params=pltpu.CompilerParams(use_tc_tiling_on_sc=False),
  )


def _split_body(x_ref, out_ref):
    out_ref[0] = x_ref[:, :HALF]
    out_ref[1] = x_ref[:, HALF:]


def _split_halves(x):
    nb = 2000
    return pl.pallas_call(
        _split_body,
        grid=(N_NODES // nb,),
        in_specs=[pl.BlockSpec((nb, HIDDEN), lambda i: (i, 0))],
        out_specs=pl.BlockSpec((NC, nb, HALF), lambda i: (0, i, 0)),
        out_shape=jax.ShapeDtypeStruct((NC, N_NODES, HALF), jnp.float32),
    )(x)


def _extract_body(ei_ref, src_ref, dst_ref):
    src_ref[...] = ei_ref[0]
    dst_ref[...] = ei_ref[1]


def _extract_indices(ei):
    return pl.pallas_call(
        _extract_body,
        out_shape=[jax.ShapeDtypeStruct((N_EDGES,), jnp.int32),
                   jax.ShapeDtypeStruct((N_EDGES,), jnp.int32)],
    )(ei)


def _node_body(x_ref, p_ref, w1_ref, b1_ref, w2_ref, b2_ref, g_ref, bl_ref,
               out_ref, outh_ref):
    x = x_ref[...]
    h = x + p_ref[...]
    t = jnp.dot(h, w1_ref[...], preferred_element_type=jnp.float32) + b1_ref[0]
    t = t * jax.nn.sigmoid(t)
    t = jnp.dot(t, w2_ref[...], preferred_element_type=jnp.float32) + b2_ref[0]
    y = x + t
    mu = jnp.mean(y, axis=1, keepdims=True)
    var = jnp.mean((y - mu) ** 2, axis=1, keepdims=True)
    y = (y - mu) * lax.rsqrt(var + 1e-5) * g_ref[0] + bl_ref[0]
    y = y * jax.nn.sigmoid(y)
    out_ref[...] = y
    outh_ref[0] = y[:, :HALF]
    outh_ref[1] = y[:, HALF:]


def _node_update(x, parts, w1, b1, w2, b2, g, bl):
    nb = 2000
    grid = (N_NODES // nb,)
    vec = lambda a: a.reshape(1, HIDDEN)
    return pl.pallas_call(
        _node_body,
        grid=grid,
        in_specs=[
            pl.BlockSpec((nb, HIDDEN), lambda i: (i, 0)),
            pl.BlockSpec((nb, HIDDEN), lambda i: (i, 0)),
            pl.BlockSpec((HIDDEN, HIDDEN), lambda i: (0, 0)),
            pl.BlockSpec((1, HIDDEN), lambda i: (0, 0)),
            pl.BlockSpec((HIDDEN, HIDDEN), lambda i: (0, 0)),
            pl.BlockSpec((1, HIDDEN), lambda i: (0, 0)),
            pl.BlockSpec((1, HIDDEN), lambda i: (0, 0)),
            pl.BlockSpec((1, HIDDEN), lambda i: (0, 0)),
        ],
        out_specs=[pl.BlockSpec((nb, HIDDEN), lambda i: (i, 0)),
                   pl.BlockSpec((NC, nb, HALF), lambda i: (0, i, 0))],
        out_shape=[jax.ShapeDtypeStruct((N_NODES, HIDDEN), jnp.float32),
                   jax.ShapeDtypeStruct((NC, N_NODES, HALF), jnp.float32)],
    )(x, parts, w1, vec(b1), w2, vec(b2), vec(g), vec(bl))


def kernel(s, edge_index_bond, edge_attr_bond, W1, b1, W2, b2, We, be,
           ln_g, ln_b):
    src, dst = _extract_indices(edge_index_bond.astype(jnp.int32))
    ea_t = edge_attr_bond.T
    x = s
    xh = _split_halves(s)
    for l in range(NUM_LAYERS):
        ea = _project_edge_attr(ea_t, We[l], be[l])
        parts = _sc_aggregate_kernel()(xh, ea, src, dst)
        x, xh = _node_update(x, parts, W1[l], b1[l], W2[l], b2[l],
                             ln_g[l], ln_b[l])
    return x
